# baseline (speedup 1.0000x reference)
import jax
import jax.numpy as jnp
from jax.experimental import pallas as pl

D_MAX = 1.0
_INTERPRET = False

_ACOS_COEF = (1.0000000050248827, 0.16666578775688576, 0.07503730543840802,
              0.04397732046770944, 0.036501366920194935, -0.009285261113212245,
              0.11148477571675892, -0.13961942458142432, 0.12518232687214173)


def _acos(x):
    s = (1.0 - x) * 0.5
    t = jnp.sqrt(s)
    acc = jnp.full_like(s, _ACOS_COEF[-1])
    for c in _ACOS_COEF[-2::-1]:
        acc = acc * s + c
    return 2.0 * t * acc


def _body(ain_ref, aout_ref, ltri_ref, pang_ref, m_ref):
    kk = 16
    ain = ain_ref[0]
    aout = aout_ref[0]
    QI = ain.shape[0]
    JB = aout.shape[1]

    nin = jnp.sqrt(ain[:, 0:1] * ain[:, 0:1]
                   + ain[:, 1:2] * ain[:, 1:2]
                   + ain[:, 2:3] * ain[:, 2:3])
    safe_in = jnp.where(nin > 0, nin, 1.0)
    ain_n = jnp.where(nin > 0, ain / safe_in, 0.0)

    nout = jnp.sqrt(aout[0:1, :] * aout[0:1, :]
                    + aout[1:2, :] * aout[1:2, :]
                    + aout[2:3, :] * aout[2:3, :])
    safe_out = jnp.where(nout > 0, nout, 1.0)
    aout_n = jnp.where(nout > 0, aout / safe_out, 0.0)

    a_b = ain_n.astype(jnp.bfloat16).astype(jnp.float32)
    b_b = aout_n.astype(jnp.bfloat16).astype(jnp.float32)
    c = (a_b[:, 0:1] * b_b[0:1, :]
         + a_b[:, 1:2] * b_b[1:2, :]
         + a_b[:, 2:3] * b_b[2:3, :])

    x = jnp.minimum(jnp.abs(c), 1.0 - 1e-7)
    d = _acos(x)

    xw = x
    ms = []
    for t in range(kk):
        m = jnp.max(xw, axis=0, keepdims=True)
        ms.append(m)
        if t < kk - 1:
            xw = jnp.where(xw >= m, -1.0, xw)

    def cnt(thr):
        return jnp.sum((x >= thr).astype(jnp.float32), axis=0, keepdims=True)

    lo = jnp.zeros((1, JB), jnp.float32)
    hi = jnp.full((1, JB), float(kk - 1), jnp.float32)
    T = ms[kk - 1]
    for _ in range(4):
        mid = jnp.floor((lo + hi) * 0.5)
        thr = ms[0]
        for t in range(1, kk):
            thr = jnp.where(mid == float(t), ms[t], thr)
        good = cnt(thr) >= float(kk)
        T = jnp.where(good, jnp.maximum(T, thr), T)
        hi = jnp.where(good, mid, hi)
        lo = jnp.where(good, lo, mid + 1.0)

    gt = (x > T).astype(jnp.float32)
    eq = (x == T).astype(jnp.float32)
    n_more = jnp.sum(gt, axis=0, keepdims=True)
    need = float(kk) - n_more
    pe = jnp.dot(ltri_ref[...], eq, preferred_element_type=jnp.float32)
    sel = gt + eq * (pe < need).astype(jnp.float32)
    msk = sel * (d <= D_MAX).astype(jnp.float32)

    bv = nout - nin
    pang_ref[0] = jnp.stack([d, bv], axis=-1).reshape(QI, 2 * JB)
    m_ref[0] = msk


def kernel(ang_in, ang_out, k_max):
    B, q_in, _ = ang_in.shape
    q_out = ang_out.shape[1]
    JB = 128
    aout_t = jnp.transpose(ang_out, (0, 2, 1))
    ii = jax.lax.broadcasted_iota(jnp.int32, (q_in, q_in), 0)
    jj = jax.lax.broadcasted_iota(jnp.int32, (q_in, q_in), 1)
    ltri = (jj < ii).astype(jnp.float32)

    pang, mask = pl.pallas_call(
        _body,
        grid=(B, q_out // JB),
        in_specs=[
            pl.BlockSpec((1, q_in, 3), lambda b, j: (b, 0, 0)),
            pl.BlockSpec((1, 3, JB), lambda b, j: (b, 0, j)),
            pl.BlockSpec((q_in, q_in), lambda b, j: (0, 0)),
        ],
        out_specs=[
            pl.BlockSpec((1, q_in, 2 * JB), lambda b, j: (b, 0, j)),
            pl.BlockSpec((1, q_in, JB), lambda b, j: (b, 0, j)),
        ],
        out_shape=[
            jax.ShapeDtypeStruct((B, q_in, 2 * q_out), jnp.float32),
            jax.ShapeDtypeStruct((B, q_in, q_out), jnp.float32),
        ],
        interpret=_INTERPRET,
    )(ang_in, aout_t, ltri)

    return pang.reshape(B, q_in, q_out, 2), mask


# speedup vs baseline: 9.2308x; 9.2308x over previous
import jax
import jax.numpy as jnp
from jax.experimental import pallas as pl

D_MAX = 1.0
_INTERPRET = False

_ACOS_COEF = (1.0000000050248827, 0.16666578775688576, 0.07503730543840802,
              0.04397732046770944, 0.036501366920194935, -0.009285261113212245,
              0.11148477571675892, -0.13961942458142432, 0.12518232687214173)


def _acos(x):
    s = (1.0 - x) * 0.5
    t = jnp.sqrt(s)
    acc = jnp.full_like(s, _ACOS_COEF[-1])
    for c in _ACOS_COEF[-2::-1]:
        acc = acc * s + c
    return 2.0 * t * acc


def _body(ain_ref, aout_ref, aout2_ref, ltri_ref, evenm_ref, pang_ref, m_ref):
    kk = 16
    ain = ain_ref[0]
    aout = aout_ref[0]
    aout2 = aout2_ref[0]
    QI = ain.shape[0]
    JB = aout.shape[1]

    nin = jnp.sqrt(ain[:, 0:1] * ain[:, 0:1]
                   + ain[:, 1:2] * ain[:, 1:2]
                   + ain[:, 2:3] * ain[:, 2:3])
    safe_in = jnp.where(nin > 0, nin, 1.0)
    ain_n = jnp.where(nin > 0, ain / safe_in, 0.0)

    nout = jnp.sqrt(aout[0:1, :] * aout[0:1, :]
                    + aout[1:2, :] * aout[1:2, :]
                    + aout[2:3, :] * aout[2:3, :])
    safe_out = jnp.where(nout > 0, nout, 1.0)
    aout_n = jnp.where(nout > 0, aout / safe_out, 0.0)

    a_b = ain_n.astype(jnp.bfloat16).astype(jnp.float32)
    b_b = aout_n.astype(jnp.bfloat16).astype(jnp.float32)
    c = (a_b[:, 0:1] * b_b[0:1, :]
         + a_b[:, 1:2] * b_b[1:2, :]
         + a_b[:, 2:3] * b_b[2:3, :])

    x = jnp.minimum(jnp.abs(c), 1.0 - 1e-7)
    d = _acos(x)

    xw = x
    ms = []
    for t in range(kk):
        m = jnp.max(xw, axis=0, keepdims=True)
        ms.append(m)
        if t < kk - 1:
            xw = jnp.where(xw >= m, -1.0, xw)

    def cnt(thr):
        return jnp.sum((x >= thr).astype(jnp.float32), axis=0, keepdims=True)

    lo = jnp.zeros((1, JB), jnp.float32)
    hi = jnp.full((1, JB), float(kk - 1), jnp.float32)
    T = ms[kk - 1]
    for _ in range(4):
        mid = jnp.floor((lo + hi) * 0.5)
        thr = ms[0]
        for t in range(1, kk):
            thr = jnp.where(mid == float(t), ms[t], thr)
        good = cnt(thr) >= float(kk)
        T = jnp.where(good, jnp.maximum(T, thr), T)
        hi = jnp.where(good, mid, hi)
        lo = jnp.where(good, lo, mid + 1.0)

    gt = (x > T).astype(jnp.float32)
    eq = (x == T).astype(jnp.float32)
    n_more = jnp.sum(gt, axis=0, keepdims=True)
    need = float(kk) - n_more
    pe = jnp.dot(ltri_ref[...], eq, preferred_element_type=jnp.float32)
    sel = gt + eq * (pe < need).astype(jnp.float32)
    msk = sel * (d <= D_MAX).astype(jnp.float32)

    m_ref[0] = msk

    nout2 = jnp.sqrt(aout2[0:1, :] * aout2[0:1, :]
                     + aout2[1:2, :] * aout2[1:2, :]
                     + aout2[2:3, :] * aout2[2:3, :])
    safe_out2 = jnp.where(nout2 > 0, nout2, 1.0)
    aout2_n = jnp.where(nout2 > 0, aout2 / safe_out2, 0.0)
    b2 = aout2_n.astype(jnp.bfloat16).astype(jnp.float32)
    c2 = (a_b[:, 0:1] * b2[0:1, :]
          + a_b[:, 1:2] * b2[1:2, :]
          + a_b[:, 2:3] * b2[2:3, :])
    d2 = _acos(jnp.minimum(jnp.abs(c2), 1.0 - 1e-7))
    bv2 = nout2 - nin
    pang_ref[0] = jnp.where(evenm_ref[...] > 0, d2, bv2)


def kernel(ang_in, ang_out, k_max):
    B, q_in, _ = ang_in.shape
    q_out = ang_out.shape[1]
    JB = 128
    aout_t = jnp.transpose(ang_out, (0, 2, 1))
    aout_dup = jnp.repeat(aout_t, 2, axis=2)
    ii = jax.lax.broadcasted_iota(jnp.int32, (q_in, q_in), 0)
    jj = jax.lax.broadcasted_iota(jnp.int32, (q_in, q_in), 1)
    ltri = (jj < ii).astype(jnp.float32)
    lane = jax.lax.broadcasted_iota(jnp.int32, (q_in, 2 * JB), 1)
    evenm = (lane % 2 == 0).astype(jnp.float32)

    pang, mask = pl.pallas_call(
        _body,
        grid=(B, q_out // JB),
        in_specs=[
            pl.BlockSpec((1, q_in, 3), lambda b, j: (b, 0, 0)),
            pl.BlockSpec((1, 3, JB), lambda b, j: (b, 0, j)),
            pl.BlockSpec((1, 3, 2 * JB), lambda b, j: (b, 0, j)),
            pl.BlockSpec((q_in, q_in), lambda b, j: (0, 0)),
            pl.BlockSpec((q_in, 2 * JB), lambda b, j: (0, 0)),
        ],
        out_specs=[
            pl.BlockSpec((1, q_in, 2 * JB), lambda b, j: (b, 0, j)),
            pl.BlockSpec((1, q_in, JB), lambda b, j: (b, 0, j)),
        ],
        out_shape=[
            jax.ShapeDtypeStruct((B, q_in, 2 * q_out), jnp.float32),
            jax.ShapeDtypeStruct((B, q_in, q_out), jnp.float32),
        ],
        interpret=_INTERPRET,
    )(ang_in, aout_t, aout_dup, ltri, evenm)

    return pang.reshape(B, q_in, q_out, 2), mask
